# baseline (speedup 1.0000x reference)
import dataclasses

import jax
import jax.numpy as jnp
from jax import lax
from jax.experimental import pallas as pl
from jax.experimental.pallas import tpu as pltpu
from jax.experimental.pallas import tpu_sc as plsc

NUM_EDGES = 320000
IN_DIM = 128
OUT_DIM = 128
NUM_ADJ = 2560000

NC, NS, L = 2, 16, 16
EPW = NUM_ADJ // NS
BLK = 1280
NBLK = EPW // BLK
NV = BLK // L
FIRE = 64
FSH = 6
FMSK = FIRE - 1
NFIRE = BLK // FIRE
CHUNK = 12800
NCHUNK = NUM_EDGES // CHUNK
RPW = CHUNK // NS

_MM_ROWS = 1280
_NMB = NUM_EDGES // _MM_ROWS


def _matmul_body(x_ref, w_ref, b_ref, o_ref):
    o_ref[...] = jnp.dot(x_ref[...], w_ref[...],
                         preferred_element_type=jnp.float32) + b_ref[...]


def _matmul_lu(h_edges, W, b):
    return pl.pallas_call(
        _matmul_body,
        grid=(2 * _NMB,),
        in_specs=[
            pl.BlockSpec((_MM_ROWS, IN_DIM), lambda j: (j % _NMB, 0)),
            pl.BlockSpec((IN_DIM, OUT_DIM), lambda j: (0, 2 * (j // _NMB))),
            pl.BlockSpec((1, OUT_DIM), lambda j: (0, 2 * (j // _NMB))),
        ],
        out_specs=pl.BlockSpec((_MM_ROWS, OUT_DIM), lambda j: (j, 0)),
        out_shape=jax.ShapeDtypeStruct((2 * NUM_EDGES, OUT_DIM), jnp.float32),
    )(h_edges, W, b.reshape(1, -1))


def _matmul_i(h_edges, W, b):
    return pl.pallas_call(
        _matmul_body,
        grid=(_NMB,),
        in_specs=[
            pl.BlockSpec((_MM_ROWS, IN_DIM), lambda i: (i, 0)),
            pl.BlockSpec((IN_DIM, OUT_DIM), lambda i: (0, 1)),
            pl.BlockSpec((1, OUT_DIM), lambda i: (0, 1)),
        ],
        out_specs=pl.BlockSpec((_MM_ROWS, OUT_DIM), lambda i: (i, 0)),
        out_shape=jax.ShapeDtypeStruct((NUM_EDGES, OUT_DIM), jnp.float32),
    )(h_edges, W, b.reshape(1, -1))


def _finish_body(a_ref, b_ref, c_ref, o_ref):
    o_ref[...] = jnp.tanh(a_ref[...] + b_ref[...] + c_ref[...])


def _finish(conv_all, h_i):
    spec = pl.BlockSpec((_MM_ROWS, OUT_DIM), lambda i: (i, 0))
    upper_spec = pl.BlockSpec((_MM_ROWS, OUT_DIM), lambda i: (i + _NMB, 0))
    return pl.pallas_call(
        _finish_body,
        grid=(_NMB,),
        in_specs=[spec, spec, upper_spec],
        out_specs=spec,
        out_shape=jax.ShapeDtypeStruct((NUM_EDGES, OUT_DIM), jnp.float32),
    )(conv_all, h_i, conv_all)


def _sc_body(hlu, eid, out_all,
             acc, ebA, ebB, cs, ct, cw, gbuf, sem, semA, semB):
    c = lax.axis_index("c")
    s = lax.axis_index("s")
    ebase = c * NUM_ADJ + s * EPW
    hoff = c * NUM_EDGES
    zi = jnp.zeros((L,), jnp.int32)
    zf = jnp.zeros((L,), jnp.float32)
    iota = lax.iota(jnp.int32, L)

    @pl.loop(0, NFIRE)
    def _(f):
        @pl.loop(0, FIRE // L)
        def _(v):
            sl = pl.ds(v * L, L)
            cs[f, sl] = zi
            ct[f, sl] = zi
            cw[f, sl] = zf

    @pl.loop(0, NCHUNK)
    def _(k):
        t0 = k * CHUNK

        @pl.loop(0, FIRE)
        def _(r):
            for v in range(OUT_DIM // L):
                gbuf[r, pl.ds(v * L, L)] = zf

        for q in range(RPW // FIRE):
            pltpu.sync_copy(gbuf, acc.at[pl.ds(s * RPW + q * FIRE, FIRE)])
        rem = RPW % FIRE
        if rem:
            pltpu.sync_copy(gbuf.at[pl.ds(0, rem)],
                            acc.at[pl.ds(s * RPW + (RPW // FIRE) * FIRE, rem)])
        plsc.subcore_barrier()

        MAXOFF = NC * NUM_ADJ - BLK

        def issue(b, eb, sm):
            off = pl.multiple_of(jnp.minimum(ebase + b * BLK, MAXOFF), 128)
            pltpu.async_copy(eid.at[:, pl.ds(off, BLK)], eb, sm)

        def wait(b, eb, sm):
            off = pl.multiple_of(jnp.minimum(ebase + b * BLK, MAXOFF), 128)
            pltpu.make_async_copy(eid.at[:, pl.ds(off, BLK)], eb, sm).wait()

        def process(eb):
            def vbody(v, base):
                sl = pl.ds(v * L, L)
                d = eb[0, sl] - t0
                m = plsc.bitcast(d, jnp.uint32) < jnp.uint32(CHUNK)
                mi = m.astype(jnp.int32)
                pos = base + plsc.cumsum(mi) - 1
                row = jnp.right_shift(pos, FSH)
                col = jnp.bitwise_and(pos, FMSK)
                plsc.store_scatter(cs, [row, col], eb[1, sl] + hoff, mask=m)
                plsc.store_scatter(ct, [row, col], d, mask=m)
                plsc.store_scatter(cw, [row, col],
                                   plsc.bitcast(eb[2, sl], jnp.float32), mask=m)
                return base + jnp.sum(mi)

            total = plsc.parallel_loop(0, NV, unroll=8, carry=jnp.int32(0))(vbody)

            nfires = jnp.right_shift(total + FIRE - 1, FSH)
            vstart = jnp.right_shift(total, 4)
            vend = nfires * (FIRE // L)

            def zerow(v, carry):
                g = v * L + iota
                row = jnp.right_shift(g, FSH)
                col = jnp.bitwise_and(g, FMSK)
                plsc.store_scatter(cw, [row, col], zf, mask=g >= total)
                return carry

            lax.fori_loop(vstart, vend, zerow, jnp.int32(0))

            def fire(f, carry):
                pltpu.sync_copy(hlu.at[cs.at[f]], gbuf)
                fv = jnp.full((L,), f, jnp.int32)

                @plsc.parallel_loop(0, FIRE, unroll=8)
                def _(r):
                    wv = plsc.load_gather(cw, [fv, jnp.full((L,), r, jnp.int32)])
                    for v in range(OUT_DIM // L):
                        sl = pl.ds(v * L, L)
                        gbuf[r, sl] = gbuf[r, sl] * wv

                pltpu.sync_copy(gbuf, acc.at[ct.at[f]], add=True)
                return carry

            lax.fori_loop(0, nfires, fire, jnp.int32(0))

        issue(jnp.int32(0), ebA, semA)

        def bpair(i, carry):
            b0 = 2 * i
            issue(b0 + 1, ebB, semB)
            wait(b0, ebA, semA)
            process(ebA)
            issue(b0 + 2, ebA, semA)
            wait(b0 + 1, ebB, semB)
            process(ebB)
            return carry

        lax.fori_loop(0, NBLK // 2, bpair, jnp.int32(0))
        wait(jnp.int32(NBLK - 1), ebA, semA)
        process(ebA)

        plsc.subcore_barrier()
        rbase = hoff + t0 + s * RPW
        pltpu.sync_copy(acc.at[pl.ds(s * RPW, RPW)], out_all.at[pl.ds(rbase, RPW)])
        plsc.subcore_barrier()


def _sc_conv(h_lu, eid):
    mesh = plsc.VectorSubcoreMesh(core_axis_name="c", subcore_axis_name="s")
    cp = pltpu.CompilerParams()
    if "needs_layout_passes" in pltpu.CompilerParams.__dataclass_fields__:
        cp = dataclasses.replace(cp, needs_layout_passes=False)
    f = pl.kernel(
        _sc_body,
        out_type=jax.ShapeDtypeStruct((2 * NUM_EDGES, OUT_DIM), jnp.float32),
        mesh=mesh,
        scratch_types=[
            pltpu.VMEM_SHARED((CHUNK, OUT_DIM), jnp.float32),
            pltpu.VMEM((3, BLK), jnp.int32),
            pltpu.VMEM((3, BLK), jnp.int32),
            pltpu.VMEM((NFIRE, FIRE), jnp.int32),
            pltpu.VMEM((NFIRE, FIRE), jnp.int32),
            pltpu.VMEM((NFIRE, FIRE), jnp.float32),
            pltpu.VMEM((FIRE, OUT_DIM), jnp.float32),
            pltpu.SemaphoreType.DMA,
            pltpu.SemaphoreType.DMA,
            pltpu.SemaphoreType.DMA,
        ],
        compiler_params=cp,
    )
    return f(h_lu, eid)


def kernel(h_edges, edge_laplacian_lower_idxs, edge_laplacian_lower_weights,
           edge_laplacian_upper_idxs, edge_laplacian_upper_weights, W, b):
    h_lu = _matmul_lu(h_edges, W, b)
    src_all = jnp.concatenate([edge_laplacian_lower_idxs[0],
                               edge_laplacian_upper_idxs[0]])
    tgt_all = jnp.concatenate([edge_laplacian_lower_idxs[1],
                               edge_laplacian_upper_idxs[1]])
    w_all = jnp.concatenate([edge_laplacian_lower_weights,
                             edge_laplacian_upper_weights])
    eid = jnp.stack([tgt_all, src_all,
                     lax.bitcast_convert_type(w_all, jnp.int32)])
    conv_all = _sc_conv(h_lu, eid)
    h_i = _matmul_i(h_edges, W, b)
    return _finish(conv_all, h_i)


# speedup vs baseline: 1.0104x; 1.0104x over previous
import dataclasses

import jax
import jax.numpy as jnp
from jax import lax
from jax.experimental import pallas as pl
from jax.experimental.pallas import tpu as pltpu
from jax.experimental.pallas import tpu_sc as plsc

NUM_EDGES = 320000
IN_DIM = 128
OUT_DIM = 128
NUM_ADJ = 2560000

NC, NS, L = 2, 16, 16
EPW = NUM_ADJ // NS
BLK = 1280
NBLK = EPW // BLK
NV = BLK // L
FIRE = 64
FSH = 6
FMSK = FIRE - 1
NFIRE = BLK // FIRE
CHUNK = 12800
NCHUNK = NUM_EDGES // CHUNK
RPW = CHUNK // NS

_MM_ROWS = 1280
_NMB = NUM_EDGES // _MM_ROWS


def _matmul_body(x_ref, w_ref, b_ref, o_ref):
    o_ref[...] = jnp.dot(x_ref[...], w_ref[...],
                         preferred_element_type=jnp.float32) + b_ref[...]


def _matmul_lu(h_edges, W, b):
    return pl.pallas_call(
        _matmul_body,
        grid=(2 * _NMB,),
        in_specs=[
            pl.BlockSpec((_MM_ROWS, IN_DIM), lambda j: (j % _NMB, 0)),
            pl.BlockSpec((IN_DIM, OUT_DIM), lambda j: (0, 2 * (j // _NMB))),
            pl.BlockSpec((1, OUT_DIM), lambda j: (0, 2 * (j // _NMB))),
        ],
        out_specs=pl.BlockSpec((_MM_ROWS, OUT_DIM), lambda j: (j, 0)),
        out_shape=jax.ShapeDtypeStruct((2 * NUM_EDGES, OUT_DIM), jnp.float32),
    )(h_edges, W, b.reshape(1, -1))


def _matmul_i(h_edges, W, b):
    return pl.pallas_call(
        _matmul_body,
        grid=(_NMB,),
        in_specs=[
            pl.BlockSpec((_MM_ROWS, IN_DIM), lambda i: (i, 0)),
            pl.BlockSpec((IN_DIM, OUT_DIM), lambda i: (0, 1)),
            pl.BlockSpec((1, OUT_DIM), lambda i: (0, 1)),
        ],
        out_specs=pl.BlockSpec((_MM_ROWS, OUT_DIM), lambda i: (i, 0)),
        out_shape=jax.ShapeDtypeStruct((NUM_EDGES, OUT_DIM), jnp.float32),
    )(h_edges, W, b.reshape(1, -1))


def _finish_body(a_ref, b_ref, c_ref, o_ref):
    o_ref[...] = jnp.tanh(a_ref[...] + b_ref[...] + c_ref[...])


def _finish(conv_all, h_i):
    spec = pl.BlockSpec((_MM_ROWS, OUT_DIM), lambda i: (i, 0))
    upper_spec = pl.BlockSpec((_MM_ROWS, OUT_DIM), lambda i: (i + _NMB, 0))
    return pl.pallas_call(
        _finish_body,
        grid=(_NMB,),
        in_specs=[spec, spec, upper_spec],
        out_specs=spec,
        out_shape=jax.ShapeDtypeStruct((NUM_EDGES, OUT_DIM), jnp.float32),
    )(conv_all, h_i, conv_all)


def _sc_body(hlu, eid, out_all,
             acc, ebA, ebB, cs, ct, cw, gbuf, sem, semA, semB):
    c = lax.axis_index("c")
    s = lax.axis_index("s")
    ebase = c * NUM_ADJ + s * EPW
    hoff = c * NUM_EDGES
    zi = jnp.zeros((L,), jnp.int32)
    zf = jnp.zeros((L,), jnp.float32)
    iota = lax.iota(jnp.int32, L)

    @pl.loop(0, NFIRE)
    def _(f):
        @pl.loop(0, FIRE // L)
        def _(v):
            sl = pl.ds(v * L, L)
            cs[f, sl] = zi
            ct[f, sl] = zi
            cw[f, sl] = zf

    @pl.loop(0, NCHUNK)
    def _(k):
        t0 = k * CHUNK

        @pl.loop(0, FIRE)
        def _(r):
            for v in range(OUT_DIM // L):
                gbuf[r, pl.ds(v * L, L)] = zf

        for q in range(RPW // FIRE):
            pltpu.sync_copy(gbuf, acc.at[pl.ds(s * RPW + q * FIRE, FIRE)])
        rem = RPW % FIRE
        if rem:
            pltpu.sync_copy(gbuf.at[pl.ds(0, rem)],
                            acc.at[pl.ds(s * RPW + (RPW // FIRE) * FIRE, rem)])
        plsc.subcore_barrier()

        MAXOFF = NC * NUM_ADJ - BLK

        def issue(b, eb, sm):
            off = pl.multiple_of(jnp.minimum(ebase + b * BLK, MAXOFF), 128)
            pltpu.async_copy(eid.at[:, pl.ds(off, BLK)], eb, sm)

        def wait(b, eb, sm):
            off = pl.multiple_of(jnp.minimum(ebase + b * BLK, MAXOFF), 128)
            pltpu.make_async_copy(eid.at[:, pl.ds(off, BLK)], eb, sm).wait()

        def process(eb):
            def vbody(v, base):
                sl = pl.ds(v * L, L)
                d = eb[0, sl] - t0
                m = plsc.bitcast(d, jnp.uint32) < jnp.uint32(CHUNK)
                mi = m.astype(jnp.int32)
                pos = base + plsc.cumsum(mi) - 1
                row = jnp.right_shift(pos, FSH)
                col = jnp.bitwise_and(pos, FMSK)
                plsc.store_scatter(cs, [row, col], eb[1, sl] + hoff, mask=m)
                plsc.store_scatter(ct, [row, col], d, mask=m)
                plsc.store_scatter(cw, [row, col],
                                   plsc.bitcast(eb[2, sl], jnp.float32), mask=m)
                return base + jnp.sum(mi)

            total = plsc.parallel_loop(0, NV, unroll=4, carry=jnp.int32(0))(vbody)

            nfires = jnp.right_shift(total + FIRE - 1, FSH)
            vstart = jnp.right_shift(total, 4)
            vend = nfires * (FIRE // L)

            def zerow(v, carry):
                g = v * L + iota
                row = jnp.right_shift(g, FSH)
                col = jnp.bitwise_and(g, FMSK)
                plsc.store_scatter(cw, [row, col], zf, mask=g >= total)
                return carry

            lax.fori_loop(vstart, vend, zerow, jnp.int32(0))

            def fire(f, carry):
                pltpu.sync_copy(hlu.at[cs.at[f]], gbuf)
                fv = jnp.full((L,), f, jnp.int32)

                @plsc.parallel_loop(0, FIRE, unroll=8)
                def _(r):
                    wv = plsc.load_gather(cw, [fv, jnp.full((L,), r, jnp.int32)])
                    for v in range(OUT_DIM // L):
                        sl = pl.ds(v * L, L)
                        gbuf[r, sl] = gbuf[r, sl] * wv

                pltpu.sync_copy(gbuf, acc.at[ct.at[f]], add=True)
                return carry

            lax.fori_loop(0, nfires, fire, jnp.int32(0))

        issue(jnp.int32(0), ebA, semA)

        def bpair(i, carry):
            b0 = 2 * i
            issue(b0 + 1, ebB, semB)
            wait(b0, ebA, semA)
            process(ebA)
            issue(b0 + 2, ebA, semA)
            wait(b0 + 1, ebB, semB)
            process(ebB)
            return carry

        lax.fori_loop(0, NBLK // 2, bpair, jnp.int32(0))
        wait(jnp.int32(NBLK - 1), ebA, semA)
        process(ebA)

        plsc.subcore_barrier()
        rbase = hoff + t0 + s * RPW
        pltpu.sync_copy(acc.at[pl.ds(s * RPW, RPW)], out_all.at[pl.ds(rbase, RPW)])
        plsc.subcore_barrier()


def _sc_conv(h_lu, eid):
    mesh = plsc.VectorSubcoreMesh(core_axis_name="c", subcore_axis_name="s")
    cp = pltpu.CompilerParams()
    if "needs_layout_passes" in pltpu.CompilerParams.__dataclass_fields__:
        cp = dataclasses.replace(cp, needs_layout_passes=False)
    f = pl.kernel(
        _sc_body,
        out_type=jax.ShapeDtypeStruct((2 * NUM_EDGES, OUT_DIM), jnp.float32),
        mesh=mesh,
        scratch_types=[
            pltpu.VMEM_SHARED((CHUNK, OUT_DIM), jnp.float32),
            pltpu.VMEM((3, BLK), jnp.int32),
            pltpu.VMEM((3, BLK), jnp.int32),
            pltpu.VMEM((NFIRE, FIRE), jnp.int32),
            pltpu.VMEM((NFIRE, FIRE), jnp.int32),
            pltpu.VMEM((NFIRE, FIRE), jnp.float32),
            pltpu.VMEM((FIRE, OUT_DIM), jnp.float32),
            pltpu.SemaphoreType.DMA,
            pltpu.SemaphoreType.DMA,
            pltpu.SemaphoreType.DMA,
        ],
        compiler_params=cp,
    )
    return f(h_lu, eid)


def kernel(h_edges, edge_laplacian_lower_idxs, edge_laplacian_lower_weights,
           edge_laplacian_upper_idxs, edge_laplacian_upper_weights, W, b):
    h_lu = _matmul_lu(h_edges, W, b)
    src_all = jnp.concatenate([edge_laplacian_lower_idxs[0],
                               edge_laplacian_upper_idxs[0]])
    tgt_all = jnp.concatenate([edge_laplacian_lower_idxs[1],
                               edge_laplacian_upper_idxs[1]])
    w_all = jnp.concatenate([edge_laplacian_lower_weights,
                             edge_laplacian_upper_weights])
    eid = jnp.stack([tgt_all, src_all,
                     lax.bitcast_convert_type(w_all, jnp.int32)])
    conv_all = _sc_conv(h_lu, eid)
    h_i = _matmul_i(h_edges, W, b)
    return _finish(conv_all, h_i)
